# baseline (speedup 1.0000x reference)
import jax
import jax.numpy as jnp
from jax.experimental import pallas as pl

N_NODES = 10000


def _ls_body(x_ref, o_ref):
    x = x_ref[...]
    m = jnp.max(x, axis=1, keepdims=True)
    e = jnp.exp(x - m)
    o_ref[...] = x - m - jnp.log(jnp.sum(e, axis=1, keepdims=True))


def _spline_conv(x, src, dst, u, Wk, Wr, b, num_nodes):
    xw0 = x @ Wk[0]
    xw1 = x @ Wk[1]
    msg = (1.0 - u)[:, None] * jnp.take(xw0, src, axis=0) + u[:, None] * jnp.take(xw1, src, axis=0)
    agg = jax.ops.segment_sum(msg, dst, num_segments=num_nodes)
    cnt = jax.ops.segment_sum(jnp.ones((src.shape[0],), dtype=x.dtype), dst, num_segments=num_nodes)
    agg = agg / jnp.maximum(cnt, 1.0)[:, None]
    return agg + x @ Wr + b


def kernel(x, edge_index, edge_attr, W1k, W1r, b1, W2k, W2r, b2):
    src = edge_index[0]
    dst = edge_index[1]
    u = jnp.clip(edge_attr[:, 0], 0.0, 1.0)
    h = _spline_conv(x, src, dst, u, W1k, W1r, b1, N_NODES)
    h = jax.nn.elu(h)
    out = _spline_conv(h, src, dst, u, W2k, W2r, b2, N_NODES)
    return pl.pallas_call(
        _ls_body,
        out_shape=jax.ShapeDtypeStruct(out.shape, out.dtype),
    )(out)


# speedup vs baseline: 6.7738x; 6.7738x over previous
import functools

import jax
import jax.numpy as jnp
from jax import lax
from jax.experimental import pallas as pl
from jax.experimental.pallas import tpu as pltpu
from jax.experimental.pallas import tpu_sc as plsc

N = 10000
E = 320000
DIN = 128
DH = 16
DO = 10
NC, NS, L = 2, 16, 16
NW = NC * NS
EPT = E // NW
B = 80
NB = EPT // B
NPAD = 10240
RPS = NPAD // NS
CPAD = 10240
CPS = CPAD // NS


def _sc_edge_body(with_cnt, t0, t1, srcr, dstr, ur, *refs):
    if with_cnt:
        (agg_o, cnt_o, src_v, dst_v, ub_v, rows0, rows1, ones_v, zb2, zb1,
         agg_sh, cnt_sh, sem0, sem1) = refs
    else:
        (agg_o, src_v, dst_v, ub_v, rows0, rows1, zb2,
         agg_sh, sem0, sem1) = refs
    c = lax.axis_index("c")
    s = lax.axis_index("s")
    wid = c * NS + s

    def _zrow(i, _):
        zb2[i] = jnp.zeros((L,), jnp.float32)
        return 0
    lax.fori_loop(0, RPS // 5, _zrow, 0)
    for k in range(5):
        pltpu.sync_copy(zb2, agg_sh.at[pl.ds(s * RPS + k * (RPS // 5), RPS // 5)])
    if with_cnt:
        def _zw(i, _):
            zb1[pl.ds(i * L, L)] = jnp.zeros((L,), jnp.float32)
            return 0
        lax.fori_loop(0, CPS // L, _zw, 0)
        pltpu.sync_copy(zb1, cnt_sh.at[pl.ds(s * CPS, CPS)])
        for k in range(B // L):
            ones_v[pl.ds(k * L, L)] = jnp.ones((L,), jnp.float32)
    plsc.subcore_barrier()

    pltpu.sync_copy(srcr.at[wid], src_v)
    pltpu.sync_copy(dstr.at[wid], dst_v)

    def _batch(j, carry):
        cp0 = pltpu.async_copy(t0.at[src_v.at[j]], rows0, sem0)
        cp1 = pltpu.async_copy(t1.at[src_v.at[j]], rows1, sem1)
        pltpu.sync_copy(ur.at[wid, j], ub_v)
        cp0.wait()
        cp1.wait()

        def _edge(i, _):
            ub = ub_v[i]
            r0 = rows0[i]
            r1 = rows1[i]
            rows0[i] = r0 + ub * (r1 - r0)
            return 0
        lax.fori_loop(0, B, _edge, 0)

        pltpu.sync_copy(rows0, agg_sh.at[dst_v.at[j]], add=True)
        if with_cnt:
            pltpu.sync_copy(ones_v, cnt_sh.at[dst_v.at[j]], add=True)
        return carry
    lax.fori_loop(0, NB, _batch, 0)
    plsc.subcore_barrier()

    pltpu.sync_copy(agg_sh.at[pl.ds(s * RPS, RPS)],
                    agg_o.at[c, pl.ds(s * RPS, RPS)])
    if with_cnt:
        pltpu.sync_copy(cnt_sh.at[pl.ds(s * CPS, CPS)],
                        cnt_o.at[c, pl.ds(s * CPS, CPS)])


def _make_sc_kernel(with_cnt):
    mesh = plsc.VectorSubcoreMesh(core_axis_name="c", subcore_axis_name="s")
    out_type = [jax.ShapeDtypeStruct((NC, NPAD, DH), jnp.float32)]
    scratch = [
        pltpu.VMEM((NB, B), jnp.int32),
        pltpu.VMEM((NB, B), jnp.int32),
        pltpu.VMEM((B, DH), jnp.float32),
        pltpu.VMEM((B, DH), jnp.float32),
        pltpu.VMEM((B, DH), jnp.float32),
    ]
    if with_cnt:
        out_type.append(jax.ShapeDtypeStruct((NC, CPAD), jnp.float32))
        scratch += [
            pltpu.VMEM((B,), jnp.float32),
            pltpu.VMEM((RPS // 5, DH), jnp.float32),
            pltpu.VMEM((CPS,), jnp.float32),
            pltpu.VMEM_SHARED((NPAD, DH), jnp.float32),
            pltpu.VMEM_SHARED((CPAD,), jnp.float32),
        ]
    else:
        scratch += [
            pltpu.VMEM((RPS // 5, DH), jnp.float32),
            pltpu.VMEM_SHARED((NPAD, DH), jnp.float32),
        ]
    scratch += [pltpu.SemaphoreType.DMA, pltpu.SemaphoreType.DMA]
    return pl.kernel(
        functools.partial(_sc_edge_body, with_cnt),
        out_type=out_type,
        mesh=mesh,
        scratch_types=scratch,
        compiler_params=pltpu.CompilerParams(use_tc_tiling_on_sc=False),
    )


_RB = 1000


def _ubc_body(u_ref, o_ref):
    o_ref[...] = jnp.minimum(jnp.maximum(
        jnp.broadcast_to(u_ref[...], o_ref.shape), 0.0), 1.0)


def _mm1_body(x_ref, w_ref, o_ref):
    o_ref[...] = jnp.dot(x_ref[...], w_ref[...],
                         preferred_element_type=jnp.float32)


def _mid_body(agg_ref, cnt0_ref, cnt1_ref, xr_ref, b1_ref, w2_ref,
              t0_ref, t1_ref, xr2_ref):
    a = agg_ref[0] + agg_ref[1]
    n = jnp.maximum(cnt0_ref[...] + cnt1_ref[...], 1.0)
    h = a / n + xr_ref[...] + b1_ref[...]
    h = jnp.where(h > 0, h, jnp.exp(jnp.minimum(h, 0.0)) - 1.0)
    hw = jnp.dot(h, w2_ref[...], preferred_element_type=jnp.float32)
    t0_ref[...] = hw[:, 0:DH]
    t1_ref[...] = hw[:, DH:2 * DH]
    xr2_ref[...] = hw[:, 2 * DH:3 * DH]


def _fin_body(agg_ref, cnt0_ref, cnt1_ref, xr2_ref, b2_ref, o_ref):
    a = agg_ref[0] + agg_ref[1]
    n = jnp.maximum(cnt0_ref[...] + cnt1_ref[...], 1.0)
    v = a / n + xr2_ref[...] + b2_ref[...]
    col = lax.broadcasted_iota(jnp.int32, v.shape, 1)
    vm = jnp.where(col < DO, v, -1e30)
    m = jnp.max(vm, axis=1, keepdims=True)
    lse = jnp.log(jnp.sum(jnp.exp(vm - m), axis=1, keepdims=True)) + m
    o_ref[...] = (v - lse)[:, 0:DO]


def kernel(x, edge_index, edge_attr, W1k, W1r, b1, W2k, W2r, b2):
    src = edge_index[0].astype(jnp.int32).reshape(NW, NB, B)
    dst = edge_index[1].astype(jnp.int32).reshape(NW, NB, B)
    _EB = 8000
    ubc = pl.pallas_call(
        _ubc_body,
        grid=(E // _EB,),
        in_specs=[pl.BlockSpec((_EB, 1), lambda i: (i, 0))],
        out_specs=pl.BlockSpec((_EB, DH), lambda i: (i, 0)),
        out_shape=jax.ShapeDtypeStruct((E, DH), jnp.float32),
    )(edge_attr[:, 0:1].astype(jnp.float32))
    u = ubc.reshape(NW, NB, B, DH)

    w1 = jnp.concatenate([W1k[0], W1k[1], W1r], axis=1)
    xw = pl.pallas_call(
        _mm1_body,
        grid=(N // _RB,),
        in_specs=[
            pl.BlockSpec((_RB, DIN), lambda i: (i, 0)),
            pl.BlockSpec((DIN, 3 * DH), lambda i: (0, 0)),
        ],
        out_specs=pl.BlockSpec((_RB, 3 * DH), lambda i: (i, 0)),
        out_shape=jax.ShapeDtypeStruct((N, 3 * DH), jnp.float32),
    )(x, w1)
    t0 = xw[:, 0:DH]
    t1 = xw[:, DH:2 * DH]
    xr1 = xw[:, 2 * DH:3 * DH]

    agg1, cnt = _make_sc_kernel(True)(t0, t1, src, dst, u)
    agg1 = agg1[:, :N]
    cnt0 = cnt[0, :N, None]
    cnt1 = cnt[1, :N, None]

    w2 = jnp.zeros((DH, 3 * DH), jnp.float32)
    w2 = w2.at[:, 0:DO].set(W2k[0])
    w2 = w2.at[:, DH:DH + DO].set(W2k[1])
    w2 = w2.at[:, 2 * DH:2 * DH + DO].set(W2r)
    s0, s1, xr2 = pl.pallas_call(
        _mid_body,
        grid=(N // _RB,),
        in_specs=[
            pl.BlockSpec((NC, _RB, DH), lambda i: (0, i, 0)),
            pl.BlockSpec((_RB, 1), lambda i: (i, 0)),
            pl.BlockSpec((_RB, 1), lambda i: (i, 0)),
            pl.BlockSpec((_RB, DH), lambda i: (i, 0)),
            pl.BlockSpec((1, DH), lambda i: (0, 0)),
            pl.BlockSpec((DH, 3 * DH), lambda i: (0, 0)),
        ],
        out_specs=[
            pl.BlockSpec((_RB, DH), lambda i: (i, 0)),
            pl.BlockSpec((_RB, DH), lambda i: (i, 0)),
            pl.BlockSpec((_RB, DH), lambda i: (i, 0)),
        ],
        out_shape=[
            jax.ShapeDtypeStruct((N, DH), jnp.float32),
            jax.ShapeDtypeStruct((N, DH), jnp.float32),
            jax.ShapeDtypeStruct((N, DH), jnp.float32),
        ],
    )(agg1, cnt0, cnt1, xr1, b1[None, :], w2)

    agg2 = _make_sc_kernel(False)(s0, s1, src, dst, u)[0][:, :N]

    b2p = jnp.zeros((DH,), jnp.float32).at[0:DO].set(b2)
    out = pl.pallas_call(
        _fin_body,
        grid=(N // _RB,),
        in_specs=[
            pl.BlockSpec((NC, _RB, DH), lambda i: (0, i, 0)),
            pl.BlockSpec((_RB, 1), lambda i: (i, 0)),
            pl.BlockSpec((_RB, 1), lambda i: (i, 0)),
            pl.BlockSpec((_RB, DH), lambda i: (i, 0)),
            pl.BlockSpec((1, DH), lambda i: (0, 0)),
        ],
        out_specs=pl.BlockSpec((_RB, DO), lambda i: (i, 0)),
        out_shape=jax.ShapeDtypeStruct((N, DO), jnp.float32),
    )(agg2, cnt0, cnt1, xr2, b2p[None, :])
    return out


# speedup vs baseline: 12.9823x; 1.9165x over previous
import functools

import jax
import jax.numpy as jnp
from jax import lax
from jax.experimental import pallas as pl
from jax.experimental.pallas import tpu as pltpu
from jax.experimental.pallas import tpu_sc as plsc

N = 10000
E = 320000
DIN = 128
DH = 16
DO = 10
NC, NS, L = 2, 16, 16
NW = NC * NS
EPT = E // NW
B = 80
NB = EPT // B
NPAD = 10240
RPS = NPAD // NS
CPAD = 10240
CPS = CPAD // NS


def _sc_edge_body(with_cnt, t0, t1, srcr, dstr, ur, *refs):
    if with_cnt:
        (agg_o, cnt_o, src_v, dst_v, u_v, rows0, rows1, ones_v, zb2, zb1,
         agg_sh, cnt_sh, sem0, sem1) = refs
    else:
        (agg_o, src_v, dst_v, u_v, rows0, rows1, zb2,
         agg_sh, sem0, sem1) = refs
    c = lax.axis_index("c")
    s = lax.axis_index("s")
    wid = c * NS + s

    def _zrow(i, _):
        zb2[i] = jnp.zeros((L,), jnp.float32)
        return 0
    lax.fori_loop(0, RPS // 5, _zrow, 0)
    for k in range(5):
        pltpu.sync_copy(zb2, agg_sh.at[pl.ds(s * RPS + k * (RPS // 5), RPS // 5)])
    if with_cnt:
        def _zw(i, _):
            zb1[pl.ds(i * L, L)] = jnp.zeros((L,), jnp.float32)
            return 0
        lax.fori_loop(0, CPS // L, _zw, 0)
        pltpu.sync_copy(zb1, cnt_sh.at[pl.ds(s * CPS, CPS)])
        for k in range(B // L):
            ones_v[pl.ds(k * L, L)] = jnp.ones((L,), jnp.float32)
    plsc.subcore_barrier()

    pltpu.sync_copy(srcr.at[wid], src_v)
    pltpu.sync_copy(dstr.at[wid], dst_v)
    pltpu.sync_copy(ur.at[wid], u_v)

    def _batch(j, carry):
        cp0 = pltpu.async_copy(t0.at[src_v.at[j]], rows0, sem0)
        cp1 = pltpu.async_copy(t1.at[src_v.at[j]], rows1, sem1)
        cp0.wait()
        cp1.wait()

        def _group(g, _):
            u16 = u_v[pl.ds((j * (B // L) + g) * L, L)]
            u16 = jnp.minimum(jnp.maximum(u16, 0.0), 1.0)
            for k in range(L):
                i = g * L + k
                ub = jnp.broadcast_to(u16[k], (L,))
                r0 = rows0[i]
                r1 = rows1[i]
                rows0[i] = r0 + ub * (r1 - r0)
            return 0
        lax.fori_loop(0, B // L, _group, 0)

        pltpu.sync_copy(rows0, agg_sh.at[dst_v.at[j]], add=True)
        if with_cnt:
            pltpu.sync_copy(ones_v, cnt_sh.at[dst_v.at[j]], add=True)
        return carry
    lax.fori_loop(0, NB, _batch, 0)
    plsc.subcore_barrier()

    pltpu.sync_copy(agg_sh.at[pl.ds(s * RPS, RPS)],
                    agg_o.at[c, pl.ds(s * RPS, RPS)])
    if with_cnt:
        pltpu.sync_copy(cnt_sh.at[pl.ds(s * CPS, CPS)],
                        cnt_o.at[c, pl.ds(s * CPS, CPS)])


def _make_sc_kernel(with_cnt):
    mesh = plsc.VectorSubcoreMesh(core_axis_name="c", subcore_axis_name="s")
    out_type = [jax.ShapeDtypeStruct((NC, NPAD, DH), jnp.float32)]
    scratch = [
        pltpu.VMEM((NB, B), jnp.int32),
        pltpu.VMEM((NB, B), jnp.int32),
        pltpu.VMEM((EPT,), jnp.float32),
        pltpu.VMEM((B, DH), jnp.float32),
        pltpu.VMEM((B, DH), jnp.float32),
    ]
    if with_cnt:
        out_type.append(jax.ShapeDtypeStruct((NC, CPAD), jnp.float32))
        scratch += [
            pltpu.VMEM((B,), jnp.float32),
            pltpu.VMEM((RPS // 5, DH), jnp.float32),
            pltpu.VMEM((CPS,), jnp.float32),
            pltpu.VMEM_SHARED((NPAD, DH), jnp.float32),
            pltpu.VMEM_SHARED((CPAD,), jnp.float32),
        ]
    else:
        scratch += [
            pltpu.VMEM((RPS // 5, DH), jnp.float32),
            pltpu.VMEM_SHARED((NPAD, DH), jnp.float32),
        ]
    scratch += [pltpu.SemaphoreType.DMA, pltpu.SemaphoreType.DMA]
    return pl.kernel(
        functools.partial(_sc_edge_body, with_cnt),
        out_type=out_type,
        mesh=mesh,
        scratch_types=scratch,
        compiler_params=pltpu.CompilerParams(use_tc_tiling_on_sc=False),
    )


_RB = 1000


def _mm1_body(x_ref, w_ref, o_ref):
    o_ref[...] = jnp.dot(x_ref[...], w_ref[...],
                         preferred_element_type=jnp.float32)


def _mid_body(agg_ref, cnt0_ref, cnt1_ref, xr_ref, b1_ref, w2_ref,
              t0_ref, t1_ref, xr2_ref):
    a = agg_ref[0] + agg_ref[1]
    n = jnp.maximum(cnt0_ref[...] + cnt1_ref[...], 1.0)
    h = a / n + xr_ref[...] + b1_ref[...]
    h = jnp.where(h > 0, h, jnp.exp(jnp.minimum(h, 0.0)) - 1.0)
    hw = jnp.dot(h, w2_ref[...], preferred_element_type=jnp.float32)
    t0_ref[...] = hw[:, 0:DH]
    t1_ref[...] = hw[:, DH:2 * DH]
    xr2_ref[...] = hw[:, 2 * DH:3 * DH]


def _fin_body(agg_ref, cnt0_ref, cnt1_ref, xr2_ref, b2_ref, o_ref):
    a = agg_ref[0] + agg_ref[1]
    n = jnp.maximum(cnt0_ref[...] + cnt1_ref[...], 1.0)
    v = a / n + xr2_ref[...] + b2_ref[...]
    col = lax.broadcasted_iota(jnp.int32, v.shape, 1)
    vm = jnp.where(col < DO, v, -1e30)
    m = jnp.max(vm, axis=1, keepdims=True)
    lse = jnp.log(jnp.sum(jnp.exp(vm - m), axis=1, keepdims=True)) + m
    o_ref[...] = (v - lse)[:, 0:DO]


def kernel(x, edge_index, edge_attr, W1k, W1r, b1, W2k, W2r, b2):
    src = edge_index[0].astype(jnp.int32).reshape(NW, NB, B)
    dst = edge_index[1].astype(jnp.int32).reshape(NW, NB, B)
    u = edge_attr[:, 0].astype(jnp.float32).reshape(NW, EPT)

    w1 = jnp.concatenate([W1k[0], W1k[1], W1r], axis=1)
    xw = pl.pallas_call(
        _mm1_body,
        grid=(N // _RB,),
        in_specs=[
            pl.BlockSpec((_RB, DIN), lambda i: (i, 0)),
            pl.BlockSpec((DIN, 3 * DH), lambda i: (0, 0)),
        ],
        out_specs=pl.BlockSpec((_RB, 3 * DH), lambda i: (i, 0)),
        out_shape=jax.ShapeDtypeStruct((N, 3 * DH), jnp.float32),
    )(x, w1)
    t0 = xw[:, 0:DH]
    t1 = xw[:, DH:2 * DH]
    xr1 = xw[:, 2 * DH:3 * DH]

    agg1, cnt = _make_sc_kernel(True)(t0, t1, src, dst, u)
    agg1 = agg1[:, :N]
    cnt0 = cnt[0, :N, None]
    cnt1 = cnt[1, :N, None]

    w2 = jnp.zeros((DH, 3 * DH), jnp.float32)
    w2 = w2.at[:, 0:DO].set(W2k[0])
    w2 = w2.at[:, DH:DH + DO].set(W2k[1])
    w2 = w2.at[:, 2 * DH:2 * DH + DO].set(W2r)
    s0, s1, xr2 = pl.pallas_call(
        _mid_body,
        grid=(N // _RB,),
        in_specs=[
            pl.BlockSpec((NC, _RB, DH), lambda i: (0, i, 0)),
            pl.BlockSpec((_RB, 1), lambda i: (i, 0)),
            pl.BlockSpec((_RB, 1), lambda i: (i, 0)),
            pl.BlockSpec((_RB, DH), lambda i: (i, 0)),
            pl.BlockSpec((1, DH), lambda i: (0, 0)),
            pl.BlockSpec((DH, 3 * DH), lambda i: (0, 0)),
        ],
        out_specs=[
            pl.BlockSpec((_RB, DH), lambda i: (i, 0)),
            pl.BlockSpec((_RB, DH), lambda i: (i, 0)),
            pl.BlockSpec((_RB, DH), lambda i: (i, 0)),
        ],
        out_shape=[
            jax.ShapeDtypeStruct((N, DH), jnp.float32),
            jax.ShapeDtypeStruct((N, DH), jnp.float32),
            jax.ShapeDtypeStruct((N, DH), jnp.float32),
        ],
    )(agg1, cnt0, cnt1, xr1, b1[None, :], w2)

    agg2 = _make_sc_kernel(False)(s0, s1, src, dst, u)[0][:, :N]

    b2p = jnp.zeros((DH,), jnp.float32).at[0:DO].set(b2)
    out = pl.pallas_call(
        _fin_body,
        grid=(N // _RB,),
        in_specs=[
            pl.BlockSpec((NC, _RB, DH), lambda i: (0, i, 0)),
            pl.BlockSpec((_RB, 1), lambda i: (i, 0)),
            pl.BlockSpec((_RB, 1), lambda i: (i, 0)),
            pl.BlockSpec((_RB, DH), lambda i: (i, 0)),
            pl.BlockSpec((1, DH), lambda i: (0, 0)),
        ],
        out_specs=pl.BlockSpec((_RB, DO), lambda i: (i, 0)),
        out_shape=jax.ShapeDtypeStruct((N, DO), jnp.float32),
    )(agg2, cnt0, cnt1, xr2, b2p[None, :])
    return out
